# Initial kernel scaffold; baseline (speedup 1.0000x reference)
import jax
import jax.numpy as jnp
from jax.experimental import pallas as pl


def kernel(features, edge_index, W0, al0, ar0, b0, g0, bt0, W1, al1, ar1, b1, g1, bt1):
    raise NotImplementedError("write your pallas kernel here")


# speedup vs baseline: 22.1777x; 22.1777x over previous
import jax
import jax.numpy as jnp
from jax import lax
from jax.experimental import pallas as pl
from jax.experimental.pallas import tpu as pltpu
from jax.experimental.pallas import tpu_sc as plsc

N = 10000
E = 320000
D = 128
NP = 10240
EC = E // 32
GB = 80
NG = EC // GB
f32 = jnp.float32
i32 = jnp.int32


def _proj_body(x_ref, w_ref, al_ref, ar_ref, h_ref, el_ref, er_ref, bnd_ref):
    x = x_ref[...]
    h = jnp.dot(x, w_ref[...], preferred_element_type=f32)
    h_ref[...] = h
    el = jnp.sum(h * al_ref[...], axis=1)
    er = jnp.sum(h * ar_ref[...], axis=1)
    el_ref[...] = el
    er_ref[...] = er
    bnd = jnp.maximum(jnp.max(el) + jnp.max(er), 0.0)
    bnd_ref[...] = jnp.full((16,), bnd, f32)


def _proj(x, W, al, ar):
    return pl.pallas_call(
        _proj_body,
        out_shape=[
            jax.ShapeDtypeStruct((N, D), f32),
            jax.ShapeDtypeStruct((N,), f32),
            jax.ShapeDtypeStruct((N,), f32),
            jax.ShapeDtypeStruct((16,), f32),
        ],
    )(x, W, al, ar)


def _post_body(p_ref, d_ref, b_ref, g_ref, bt_ref, o_ref):
    psum = p_ref[pl.ds(0, N), :] + p_ref[pl.ds(NP, N), :]
    dsum = d_ref[pl.ds(0, N)] + d_ref[pl.ds(NP, N)]
    t = psum / jnp.reshape(dsum + 1e-9, (N, 1)) + b_ref[...]
    mu = jnp.mean(t, axis=1, keepdims=True)
    var = jnp.mean((t - mu) ** 2, axis=1, keepdims=True)
    y = (t - mu) * lax.rsqrt(var + 1e-5) * g_ref[...] + bt_ref[...]
    o_ref[...] = jnp.where(y > 0, y, jnp.exp(y) - 1.0)


def _post(p, d, b, g, bt):
    return pl.pallas_call(
        _post_body,
        out_shape=jax.ShapeDtypeStruct((N, D), f32),
    )(p, d, b, g, bt)


def _edge_body(el_h, er_h, bnd_h, src_h, dst_h, h_h, out_h, den_h,
               el_v, er_v, bnd_v, rows, sbuf, dbuf, wtmp, zbuf,
               den_sh, acc_sh, sem):
    c = lax.axis_index("c")
    s = lax.axis_index("s")
    eoff = (c * 16 + s) * EC

    pltpu.sync_copy(el_h, el_v)
    pltpu.sync_copy(er_h, er_v)
    pltpu.sync_copy(bnd_h, bnd_v)

    z16 = jnp.zeros((16,), f32)

    @pl.loop(0, 40)
    def _(i):
        zbuf[pl.ds(i * 16, 16)] = z16

    @pl.loop(0, GB)
    def _(e):
        for r in range(8):
            rows[e, pl.ds(r * 16, 16)] = z16

    pltpu.sync_copy(zbuf, den_sh.at[pl.ds(s * 640, 640)])
    for i in range(8):
        pltpu.sync_copy(rows, acc_sh.at[pl.ds(s * 640 + i * GB, GB), :])

    plsc.subcore_barrier()

    bndv = bnd_v[...]

    @pl.loop(0, NG)
    def _(g):
        base = eoff + g * GB
        pltpu.sync_copy(src_h.at[pl.ds(base, GB)], sbuf)
        pltpu.sync_copy(dst_h.at[pl.ds(base, GB)], dbuf)

        cp = pltpu.async_copy(h_h.at[sbuf], rows, sem)

        @pl.loop(0, GB // 16)
        def _(t):
            sv = sbuf[pl.ds(t * 16, 16)]
            dv = dbuf[pl.ds(t * 16, 16)]
            av = plsc.load_gather(el_v, [sv])
            bv = plsc.load_gather(er_v, [dv])
            ev = av + bv
            lr = jnp.where(ev > 0, ev, 0.2 * ev)
            wtmp[pl.ds(t * 16, 16)] = jnp.exp(lr - bndv)

        pltpu.sync_copy(wtmp, den_sh.at[dbuf], add=True)
        cp.wait()

        @pl.loop(0, GB)
        def _(e):
            bc = plsc.load_gather(wtmp, [jnp.full((16,), 0, i32) + e])
            for r in range(8):
                rows[e, pl.ds(r * 16, 16)] = rows[e, pl.ds(r * 16, 16)] * bc

        pltpu.sync_copy(rows, acc_sh.at[dbuf], add=True)

    plsc.subcore_barrier()
    pltpu.sync_copy(acc_sh.at[pl.ds(s * 640, 640), :],
                    out_h.at[pl.ds(c * NP + s * 640, 640), :])
    pltpu.sync_copy(den_sh.at[pl.ds(s * 640, 640)],
                    den_h.at[pl.ds(c * NP + s * 640, 640)])


def _edge(el, er, bnd, src, dst, h):
    mesh = plsc.VectorSubcoreMesh(
        core_axis_name="c", subcore_axis_name="s", num_cores=2,
        num_subcores=16)
    fn = pl.kernel(
        _edge_body,
        out_type=[
            jax.ShapeDtypeStruct((2 * NP, D), f32),
            jax.ShapeDtypeStruct((2 * NP,), f32),
        ],
        mesh=mesh,
        compiler_params=pltpu.CompilerParams(needs_layout_passes=False),
        scratch_types=[
            pltpu.VMEM((N,), f32),
            pltpu.VMEM((N,), f32),
            pltpu.VMEM((16,), f32),
            pltpu.VMEM((GB, D), f32),
            pltpu.VMEM((GB,), i32),
            pltpu.VMEM((GB,), i32),
            pltpu.VMEM((GB,), f32),
            pltpu.VMEM((640,), f32),
            pltpu.VMEM_SHARED((NP,), f32),
            pltpu.VMEM_SHARED((NP, D), f32),
            pltpu.SemaphoreType.DMA,
        ],
    )
    return fn(el, er, bnd, src, dst, h)


def kernel(features, edge_index, W0, al0, ar0, b0, g0, bt0,
           W1, al1, ar1, b1, g1, bt1):
    src = edge_index[0]
    dst = edge_index[1]
    h0, el0, er0, bnd0 = _proj(features, W0, al0, ar0)
    p0, d0 = _edge(el0, er0, bnd0, src, dst, h0)
    f0 = _post(p0, d0, b0, g0, bt0)
    h1, el1, er1, bnd1 = _proj(f0, W1, al1, ar1)
    p1, d1 = _edge(el1, er1, bnd1, src, dst, h1)
    return _post(p1, d1, b1, g1, bt1)


# speedup vs baseline: 44.3789x; 2.0011x over previous
import jax
import jax.numpy as jnp
from jax import lax
from jax.experimental import pallas as pl
from jax.experimental.pallas import tpu as pltpu
from jax.experimental.pallas import tpu_sc as plsc

N = 10000
E = 320000
D = 128
NP = 10240
GB = 80
TPG = E // GB // 32
NIT = (TPG - 1) // 4
f32 = jnp.float32
i32 = jnp.int32


def _proj_body(x_ref, w_ref, al_ref, ar_ref, h_ref, el_ref, er_ref, bnd_ref):
    x = x_ref[...]
    h = jnp.dot(x, w_ref[...], preferred_element_type=f32)
    h_ref[...] = h
    el = jnp.sum(h * al_ref[...], axis=1)
    er = jnp.sum(h * ar_ref[...], axis=1)
    el_ref[...] = el
    er_ref[...] = er
    bnd = jnp.maximum(jnp.max(el) + jnp.max(er), 0.0)
    bnd_ref[...] = jnp.full((16,), bnd, f32)


def _proj(x, W, al, ar):
    return pl.pallas_call(
        _proj_body,
        out_shape=[
            jax.ShapeDtypeStruct((N, D), f32),
            jax.ShapeDtypeStruct((N,), f32),
            jax.ShapeDtypeStruct((N,), f32),
            jax.ShapeDtypeStruct((16,), f32),
        ],
    )(x, W, al, ar)


def _post_body(p_ref, d_ref, b_ref, g_ref, bt_ref, o_ref):
    psum = p_ref[pl.ds(0, N), :] + p_ref[pl.ds(NP, N), :]
    dsum = d_ref[pl.ds(0, N)] + d_ref[pl.ds(NP, N)]
    t = psum / jnp.reshape(dsum + 1e-9, (N, 1)) + b_ref[...]
    mu = jnp.mean(t, axis=1, keepdims=True)
    var = jnp.mean((t - mu) ** 2, axis=1, keepdims=True)
    y = (t - mu) * lax.rsqrt(var + 1e-5) * g_ref[...] + bt_ref[...]
    o_ref[...] = jnp.where(y > 0, y, jnp.exp(y) - 1.0)


def _post(p, d, b, g, bt):
    return pl.pallas_call(
        _post_body,
        out_shape=jax.ShapeDtypeStruct((N, D), f32),
    )(p, d, b, g, bt)


def _edge_body(el_h, er_h, bnd_h, src_h, dst_h, h_h, out_h, den_h,
               el_v, er_v, bnd_v,
               rows0, rows1,
               sb0, sb1, sb2, sb3,
               db0, db1, db2, db3,
               wb0, wb1, wb2, wb3,
               zbuf, den_sh, acc_sh,
               sg0, sg1, ss0, ss1,
               sd0, sd1, sd2, sd3,
               semi):
    c = lax.axis_index("c")
    s = lax.axis_index("s")
    base_e = (c * 16 + s) * TPG * GB

    rows = [rows0, rows1]
    sb = [sb0, sb1, sb2, sb3]
    db = [db0, db1, db2, db3]
    wb = [wb0, wb1, wb2, wb3]
    semg = [sg0, sg1]
    sems = [ss0, ss1]
    semd = [sd0, sd1, sd2, sd3]

    pltpu.sync_copy(el_h, el_v)
    pltpu.sync_copy(er_h, er_v)
    pltpu.sync_copy(bnd_h, bnd_v)

    z16 = jnp.zeros((16,), f32)

    @pl.loop(0, 40)
    def _(i):
        zbuf[pl.ds(i * 16, 16)] = z16

    @pl.loop(0, GB)
    def _(e):
        for r in range(D // 16):
            rows0[e, pl.ds(r * 16, 16)] = z16

    pltpu.sync_copy(zbuf, den_sh.at[pl.ds(s * 640, 640)])
    for i in range(8):
        pltpu.sync_copy(rows0, acc_sh.at[pl.ds(s * 640 + i * GB, GB), :])

    plsc.subcore_barrier()

    bndv = bnd_v[...]

    def compute_w(q):
        @pl.loop(0, GB // 16)
        def _(t):
            sv = sb[q][pl.ds(t * 16, 16)]
            dv = db[q][pl.ds(t * 16, 16)]
            av = plsc.load_gather(el_v, [sv])
            bv = plsc.load_gather(er_v, [dv])
            ev = av + bv
            lr = jnp.where(ev > 0, ev, 0.2 * ev)
            wb[q][pl.ds(t * 16, 16)] = jnp.exp(lr - bndv)

    def scale_rows(q, p):
        @pl.loop(0, GB // 16)
        def _(t):
            for e16 in range(16):
                e = t * 16 + e16
                bc = plsc.load_gather(wb[q], [jnp.full((16,), 0, i32) + e])
                for r in range(D // 16):
                    rows[p][e, pl.ds(r * 16, 16)] = (
                        rows[p][e, pl.ds(r * 16, 16)] * bc)

    def phase(b, q, in_loop):
        g = b * 4 + q
        p = q & 1
        pn = 1 - p
        qn = (q + 1) % 4
        qm = (q + 3) % 4
        if in_loop:
            off1 = base_e + (g + 1) * GB
            cpa = pltpu.async_copy(src_h.at[pl.ds(off1, GB)], sb[qn], semi)
            cpb = pltpu.async_copy(dst_h.at[pl.ds(off1, GB)], db[qn], semi)
        compute_w(q)
        if in_loop:
            cpa.wait()
            cpb.wait()

        def free_rows_pn():
            pltpu.make_async_copy(
                rows[pn], acc_sh.at[db[qm]], sems[pn]).wait()

        if in_loop and q == 0:
            @pl.when(b > 0)
            def _():
                free_rows_pn()
        else:
            free_rows_pn()
        if in_loop:
            pltpu.async_copy(h_h.at[sb[qn]], rows[pn], semg[pn])
        pltpu.sync_copy(wb[q], den_sh.at[db[q]], add=True)
        pltpu.make_async_copy(h_h.at[sb[q]], rows[p], semg[p]).wait()
        scale_rows(q, p)
        pltpu.async_copy(rows[p], acc_sh.at[db[q]], sems[p], add=True)

    pltpu.sync_copy(src_h.at[pl.ds(base_e, GB)], sb[0])
    pltpu.sync_copy(dst_h.at[pl.ds(base_e, GB)], db[0])
    pltpu.async_copy(h_h.at[sb[0]], rows[0], semg[0])

    @pl.loop(0, NIT)
    def _(b):
        phase(b, 0, True)
        phase(b, 1, True)
        phase(b, 2, True)
        phase(b, 3, True)

    phase(NIT, 0, False)

    pltpu.make_async_copy(rows[0], acc_sh.at[db[0]], sems[0]).wait()

    plsc.subcore_barrier()
    pltpu.sync_copy(acc_sh.at[pl.ds(s * 640, 640), :],
                    out_h.at[pl.ds(c * NP + s * 640, 640), :])
    pltpu.sync_copy(den_sh.at[pl.ds(s * 640, 640)],
                    den_h.at[pl.ds(c * NP + s * 640, 640)])


def _edge(el, er, bnd, src1, dst1, h):
    mesh = plsc.VectorSubcoreMesh(
        core_axis_name="c", subcore_axis_name="s", num_cores=2,
        num_subcores=16)
    fn = pl.kernel(
        _edge_body,
        out_type=[
            jax.ShapeDtypeStruct((2 * NP, D), f32),
            jax.ShapeDtypeStruct((2 * NP,), f32),
        ],
        mesh=mesh,
        compiler_params=pltpu.CompilerParams(needs_layout_passes=False),
        scratch_types=(
            [
                pltpu.VMEM((N,), f32),
                pltpu.VMEM((N,), f32),
                pltpu.VMEM((16,), f32),
            ]
            + [pltpu.VMEM((GB, D), f32)] * 2
            + [pltpu.VMEM((GB,), i32)] * 4
            + [pltpu.VMEM((GB,), i32)] * 4
            + [pltpu.VMEM((GB,), f32)] * 4
            + [
                pltpu.VMEM((640,), f32),
                pltpu.VMEM_SHARED((NP,), f32),
                pltpu.VMEM_SHARED((NP, D), f32),
            ]
            + [pltpu.SemaphoreType.DMA] * 9
        ),
    )
    return fn(el, er, bnd, src1, dst1, h)


def kernel(features, edge_index, W0, al0, ar0, b0, g0, bt0,
           W1, al1, ar1, b1, g1, bt1):
    src1 = edge_index[0]
    dst1 = edge_index[1]

    h0, el0, er0, bnd0 = _proj(features, W0, al0, ar0)
    p0, d0 = _edge(el0, er0, bnd0, src1, dst1, h0)
    f0 = _post(p0, d0, b0, g0, bt0)
    h1, el1, er1, bnd1 = _proj(f0, W1, al1, ar1)
    p1, d1 = _edge(el1, er1, bnd1, src1, dst1, h1)
    return _post(p1, d1, b1, g1, bt1)


# speedup vs baseline: 46.2889x; 1.0430x over previous
import jax
import jax.numpy as jnp
from jax import lax
from jax.experimental import pallas as pl
from jax.experimental.pallas import tpu as pltpu
from jax.experimental.pallas import tpu_sc as plsc

N = 10000
E = 320000
D = 128
NP = 10240
GB = 80
TPG = E // GB // 32
NIT = (TPG - 1) // 4
f32 = jnp.float32
i32 = jnp.int32


def _proj_body(x_ref, w_ref, al_ref, ar_ref, h_ref, el_ref, er_ref, bnd_ref):
    x = x_ref[...]
    h = jnp.dot(x, w_ref[...], preferred_element_type=f32)
    h_ref[...] = h
    el = jnp.sum(h * al_ref[...], axis=1)
    er = jnp.sum(h * ar_ref[...], axis=1)
    el_ref[...] = el
    er_ref[...] = er
    bnd = jnp.maximum(jnp.max(el) + jnp.max(er), 0.0)
    bnd_ref[...] = jnp.full((16,), bnd, f32)


def _proj(x, W, al, ar):
    return pl.pallas_call(
        _proj_body,
        out_shape=[
            jax.ShapeDtypeStruct((N, D), f32),
            jax.ShapeDtypeStruct((N,), f32),
            jax.ShapeDtypeStruct((N,), f32),
            jax.ShapeDtypeStruct((16,), f32),
        ],
    )(x, W, al, ar)


def _post_body(p_ref, d_ref, b_ref, g_ref, bt_ref, o_ref):
    psum = p_ref[pl.ds(0, N), :] + p_ref[pl.ds(NP, N), :]
    dsum = d_ref[pl.ds(0, N)] + d_ref[pl.ds(NP, N)]
    t = psum / jnp.reshape(dsum + 1e-9, (N, 1)) + b_ref[...]
    mu = jnp.mean(t, axis=1, keepdims=True)
    var = jnp.mean((t - mu) ** 2, axis=1, keepdims=True)
    y = (t - mu) * lax.rsqrt(var + 1e-5) * g_ref[...] + bt_ref[...]
    o_ref[...] = jnp.where(y > 0, y, jnp.exp(y) - 1.0)


def _post(p, d, b, g, bt):
    return pl.pallas_call(
        _post_body,
        out_shape=jax.ShapeDtypeStruct((N, D), f32),
    )(p, d, b, g, bt)


def _edge_body(el_h, er_h, bnd_h, src_h, dst_h, h_h, out_h, den_h,
               el_v, er_v, bnd_v,
               rows0, rows1,
               sb0, sb1, sb2, sb3,
               db0, db1, db2, db3,
               wb0, wb1, wb2, wb3,
               zbuf, den_sh, acc_sh,
               sg0, sg1, ss0, ss1,
               sd0, sd1, sd2, sd3,
               semi):
    c = lax.axis_index("c")
    s = lax.axis_index("s")
    base_e = (c * 16 + s) * TPG * GB

    rows = [rows0, rows1]
    sb = [sb0, sb1, sb2, sb3]
    db = [db0, db1, db2, db3]
    wb = [wb0, wb1, wb2, wb3]
    semg = [sg0, sg1]
    sems = [ss0, ss1]
    semd = [sd0, sd1, sd2, sd3]

    pltpu.sync_copy(el_h, el_v)
    pltpu.sync_copy(er_h, er_v)
    pltpu.sync_copy(bnd_h, bnd_v)

    z16 = jnp.zeros((16,), f32)

    @pl.loop(0, 40)
    def _(i):
        zbuf[pl.ds(i * 16, 16)] = z16

    @pl.loop(0, GB)
    def _(e):
        for r in range(D // 16):
            rows0[e, pl.ds(r * 16, 16)] = z16

    pltpu.sync_copy(zbuf, den_sh.at[pl.ds(s * 640, 640)])
    for i in range(8):
        pltpu.sync_copy(rows0, acc_sh.at[pl.ds(s * 640 + i * GB, GB), :])

    plsc.subcore_barrier()

    bndv = bnd_v[...]

    def compute_w(q):
        @pl.loop(0, GB // 16)
        def _(t):
            sv = sb[q][pl.ds(t * 16, 16)]
            dv = db[q][pl.ds(t * 16, 16)]
            av = plsc.load_gather(el_v, [sv])
            bv = plsc.load_gather(er_v, [dv])
            ev = av + bv
            lr = jnp.where(ev > 0, ev, 0.2 * ev)
            wb[q][pl.ds(t * 16, 16)] = jnp.exp(lr - bndv)

    def scale_rows(q, p):
        @pl.loop(0, GB // 16)
        def _(t):
            for e16 in range(16):
                e = t * 16 + e16
                bc = plsc.load_gather(wb[q], [jnp.full((16,), 0, i32) + e])
                for r in range(D // 16):
                    rows[p][e, pl.ds(r * 16, 16)] = (
                        rows[p][e, pl.ds(r * 16, 16)] * bc)

    def phase(b, q, in_loop):
        g = b * 4 + q
        p = q & 1
        pn = 1 - p
        qn = (q + 1) % 4
        qm = (q + 3) % 4

        def wait_den_qn():
            pltpu.make_async_copy(
                wb[qn], den_sh.at[db[qn]], semd[qn]).wait()

        if in_loop:
            if q == 3:
                wait_den_qn()
            else:
                @pl.when(b > 0)
                def _():
                    wait_den_qn()
            off1 = base_e + (g + 1) * GB
            cpa = pltpu.async_copy(src_h.at[pl.ds(off1, GB)], sb[qn], semi)
            cpb = pltpu.async_copy(dst_h.at[pl.ds(off1, GB)], db[qn], semi)
        compute_w(q)
        if in_loop:
            cpa.wait()
            cpb.wait()

        def free_rows_pn():
            pltpu.make_async_copy(
                rows[pn], acc_sh.at[db[qm]], sems[pn]).wait()

        if in_loop and q == 0:
            @pl.when(b > 0)
            def _():
                free_rows_pn()
        else:
            free_rows_pn()
        if in_loop:
            pltpu.async_copy(h_h.at[sb[qn]], rows[pn], semg[pn])
        pltpu.async_copy(wb[q], den_sh.at[db[q]], semd[q], add=True)
        pltpu.make_async_copy(h_h.at[sb[q]], rows[p], semg[p]).wait()
        scale_rows(q, p)
        pltpu.async_copy(rows[p], acc_sh.at[db[q]], sems[p], add=True)

    pltpu.sync_copy(src_h.at[pl.ds(base_e, GB)], sb[0])
    pltpu.sync_copy(dst_h.at[pl.ds(base_e, GB)], db[0])
    pltpu.async_copy(h_h.at[sb[0]], rows[0], semg[0])

    @pl.loop(0, NIT)
    def _(b):
        phase(b, 0, True)
        phase(b, 1, True)
        phase(b, 2, True)
        phase(b, 3, True)

    phase(NIT, 0, False)

    pltpu.make_async_copy(rows[0], acc_sh.at[db[0]], sems[0]).wait()
    for qq in (1, 2, 3, 0):
        pltpu.make_async_copy(
            wb[qq], den_sh.at[db[qq]], semd[qq]).wait()

    plsc.subcore_barrier()
    pltpu.sync_copy(acc_sh.at[pl.ds(s * 640, 640), :],
                    out_h.at[pl.ds(c * NP + s * 640, 640), :])
    pltpu.sync_copy(den_sh.at[pl.ds(s * 640, 640)],
                    den_h.at[pl.ds(c * NP + s * 640, 640)])


def _edge(el, er, bnd, src1, dst1, h):
    mesh = plsc.VectorSubcoreMesh(
        core_axis_name="c", subcore_axis_name="s", num_cores=2,
        num_subcores=16)
    fn = pl.kernel(
        _edge_body,
        out_type=[
            jax.ShapeDtypeStruct((2 * NP, D), f32),
            jax.ShapeDtypeStruct((2 * NP,), f32),
        ],
        mesh=mesh,
        compiler_params=pltpu.CompilerParams(needs_layout_passes=False),
        scratch_types=(
            [
                pltpu.VMEM((N,), f32),
                pltpu.VMEM((N,), f32),
                pltpu.VMEM((16,), f32),
            ]
            + [pltpu.VMEM((GB, D), f32)] * 2
            + [pltpu.VMEM((GB,), i32)] * 4
            + [pltpu.VMEM((GB,), i32)] * 4
            + [pltpu.VMEM((GB,), f32)] * 4
            + [
                pltpu.VMEM((640,), f32),
                pltpu.VMEM_SHARED((NP,), f32),
                pltpu.VMEM_SHARED((NP, D), f32),
            ]
            + [pltpu.SemaphoreType.DMA] * 9
        ),
    )
    return fn(el, er, bnd, src1, dst1, h)


def kernel(features, edge_index, W0, al0, ar0, b0, g0, bt0,
           W1, al1, ar1, b1, g1, bt1):
    src1 = edge_index[0]
    dst1 = edge_index[1]

    h0, el0, er0, bnd0 = _proj(features, W0, al0, ar0)
    p0, d0 = _edge(el0, er0, bnd0, src1, dst1, h0)
    f0 = _post(p0, d0, b0, g0, bt0)
    h1, el1, er1, bnd1 = _proj(f0, W1, al1, ar1)
    p1, d1 = _edge(el1, er1, bnd1, src1, dst1, h1)
    return _post(p1, d1, b1, g1, bt1)
